# baseline (speedup 1.0000x reference)
import functools

import jax
import jax.numpy as jnp
from jax import lax
from jax.experimental import pallas as pl
from jax.experimental.pallas import tpu as pltpu
from jax.experimental.pallas import tpu_sc as plsc

N = 10000
E = 160000
D = 128
CAP = 64

NC = 2
NS = 16
NW = NC * NS

N_PAD = 10240
ROWS_PER_W = N_PAD // NW
E_PAD = 160256
E_PER_W = E_PAD // NW
SENT = 1 << 20

FLAT = N * CAP
GCHUNK = 128
CH_PER_W = 157
NCHUNKS = NW * CH_PER_W
FLAT_PAD = NCHUNKS * GCHUNK
NBUF = 4

_SC_PARAMS = pltpu.CompilerParams(needs_layout_passes=False)


def _mesh():
    return plsc.VectorSubcoreMesh(core_axis_name="c", subcore_axis_name="s")


def _wid():
    return lax.axis_index("s") * NC + lax.axis_index("c")


def _deg_partial(cols_pad):
    @functools.partial(
        pl.kernel,
        mesh=_mesh(),
        out_type=jax.ShapeDtypeStruct((NW, N_PAD), jnp.float32),
        scratch_types=[
            pltpu.VMEM((N_PAD,), jnp.float32),
            pltpu.VMEM((E_PER_W,), jnp.int32),
        ],
        compiler_params=_SC_PARAMS,
        name="sc_deg_partial",
    )
    def k(cols_hbm, degp_hbm, hist_v, cbuf_v):
        wid = _wid()
        zeros16 = jnp.zeros((16,), jnp.float32)
        ones16 = jnp.ones((16,), jnp.float32)

        def zero_body(i, _):
            hist_v[pl.ds(i * 16, 16)] = zeros16
            return _

        lax.fori_loop(0, N_PAD // 16, zero_body, None)

        pltpu.sync_copy(cols_hbm.at[pl.ds(wid * E_PER_W, E_PER_W)], cbuf_v)

        def body(i, _):
            c = cbuf_v[pl.ds(i * 16, 16)]
            m = c < N
            plsc.addupdate_scatter(hist_v, [c], ones16, mask=m)
            return _

        lax.fori_loop(0, E_PER_W // 16, body, None)
        pltpu.sync_copy(hist_v, degp_hbm.at[wid])

    return k(cols_pad)


def _deg_merge(degp):
    def body(degp_ref, deg_ref):
        deg_ref[...] = jnp.sum(degp_ref[...], axis=0) + 1.0

    return pl.pallas_call(
        body,
        out_shape=jax.ShapeDtypeStruct((N_PAD // 128, 128), jnp.float32),
        name="tc_deg_merge",
    )(degp)


def _matmul(x, w):
    n = x.shape[0]
    blk = 400
    assert n % blk == 0

    def body(x_ref, w_ref, o_ref):
        o_ref[...] = lax.dot_general(
            x_ref[...], w_ref[...], (((1,), (0,)), ((), ())),
            preferred_element_type=jnp.float32)

    return pl.pallas_call(
        body,
        grid=(n // blk,),
        in_specs=[
            pl.BlockSpec((blk, D), lambda i: (i, 0)),
            pl.BlockSpec((D, D), lambda i: (0, 0)),
        ],
        out_specs=pl.BlockSpec((blk, D), lambda i: (i, 0)),
        out_shape=jax.ShapeDtypeStruct((n, D), jnp.float32),
        name="tc_matmul",
    )(x, w)


def _build_lists(rows_pad, cols_pad, deg_flat):
    ECHUNK = 2048
    NEC = E_PAD // ECHUNK if E_PAD % ECHUNK == 0 else E_PAD // ECHUNK + 1
    NFULL = E_PAD // ECHUNK
    TAIL = E_PAD - NFULL * ECHUNK

    @functools.partial(
        pl.kernel,
        mesh=_mesh(),
        out_type=[
            jax.ShapeDtypeStruct((N_PAD * CAP,), jnp.int32),
            jax.ShapeDtypeStruct((N_PAD * CAP,), jnp.float32),
        ],
        scratch_types=[
            pltpu.VMEM((N_PAD,), jnp.float32),
            pltpu.VMEM((ROWS_PER_W * CAP,), jnp.int32),
            pltpu.VMEM((ROWS_PER_W * CAP,), jnp.float32),
            pltpu.VMEM((ROWS_PER_W,), jnp.int32),
            pltpu.VMEM((ECHUNK,), jnp.int32),
            pltpu.VMEM((ECHUNK,), jnp.int32),
        ],
        compiler_params=_SC_PARAMS,
        name="sc_build_lists",
    )
    def k(rows_hbm, cols_hbm, deg_hbm, idx_hbm, dp_hbm,
          deg_v, idx_b, dp_b, cnt_v, rbuf, cbuf):
        wid = _wid()
        rlo = wid * ROWS_PER_W

        pltpu.sync_copy(deg_hbm, deg_v)

        zero16i = jnp.zeros((16,), jnp.int32)
        zero16f = jnp.zeros((16,), jnp.float32)
        one16i = jnp.ones((16,), jnp.int32)
        iota16 = lax.iota(jnp.int32, 16)

        def zb(i, _):
            idx_b[pl.ds(i * 16, 16)] = zero16i
            dp_b[pl.ds(i * 16, 16)] = zero16f
            return _

        lax.fori_loop(0, ROWS_PER_W * CAP // 16, zb, None)

        def init_body(i, _):
            rl = iota16 + i * 16
            rg = rl + rlo
            ok = rg < N
            d = plsc.load_gather(deg_v, [rg], mask=ok)
            plsc.store_scatter(idx_b, [rl * CAP], rg, mask=ok)
            plsc.store_scatter(dp_b, [rl * CAP], d * d, mask=ok)
            cnt_v[pl.ds(i * 16, 16)] = one16i
            return _

        lax.fori_loop(0, ROWS_PER_W // 16, init_body, None)

        def process(nvec):
            def body(j, _):
                r = rbuf[pl.ds(j * 16, 16)]
                c = cbuf[pl.ds(j * 16, 16)]
                m = (r >= rlo) & (r < rlo + ROWS_PER_W)
                rl = jnp.where(m, r - rlo, ROWS_PER_W + iota16)
                occ, lastm = plsc.scan_count(rl, mask=m)
                base = plsc.load_gather(cnt_v, [rl], mask=m)
                slot = base + occ - 1
                ok = m & (slot < CAP)
                flat = jnp.where(ok, rl * CAP + slot, 0)
                plsc.store_scatter(idx_b, [flat], c, mask=ok)
                dr = plsc.load_gather(deg_v, [r], mask=m)
                dc = plsc.load_gather(deg_v, [c], mask=m)
                plsc.store_scatter(dp_b, [flat], dr * dc, mask=ok)
                newc = jnp.minimum(base + occ, CAP)
                plsc.store_scatter(cnt_v, [rl], newc, mask=m & lastm)
                return _

            lax.fori_loop(0, nvec, body, None)

        def chunk_body(ci, _):
            off = ci * ECHUNK
            pltpu.sync_copy(rows_hbm.at[pl.ds(off, ECHUNK)], rbuf)
            pltpu.sync_copy(cols_hbm.at[pl.ds(off, ECHUNK)], cbuf)
            process(ECHUNK // 16)
            return _

        lax.fori_loop(0, NFULL, chunk_body, None)
        if TAIL:
            off = NFULL * ECHUNK
            pltpu.sync_copy(rows_hbm.at[pl.ds(off, TAIL)],
                            rbuf.at[pl.ds(0, TAIL)])
            pltpu.sync_copy(cols_hbm.at[pl.ds(off, TAIL)],
                            cbuf.at[pl.ds(0, TAIL)])
            process(TAIL // 16)

        pltpu.sync_copy(idx_b, idx_hbm.at[pl.ds(rlo * CAP, ROWS_PER_W * CAP)])
        pltpu.sync_copy(dp_b, dp_hbm.at[pl.ds(rlo * CAP, ROWS_PER_W * CAP)])

    return k(rows_pad, cols_pad, deg_flat)


def _gather_rows(nbr_idx_flat, h):
    OUTER = CH_PER_W // NBUF
    TAIL = CH_PER_W - OUTER * NBUF

    @functools.partial(
        pl.kernel,
        mesh=_mesh(),
        out_type=jax.ShapeDtypeStruct((FLAT_PAD, D), jnp.float32),
        scratch_types=[
            pltpu.VMEM((CH_PER_W * GCHUNK,), jnp.int32),
            [pltpu.VMEM((GCHUNK, D), jnp.float32) for _ in range(NBUF)],
            [pltpu.SemaphoreType.DMA for _ in range(NBUF)],
        ],
        compiler_params=_SC_PARAMS,
        name="sc_gather_rows",
    )
    def k(idx_hbm, h_hbm, out_hbm, idx_v, bufs, sems):
        wid = _wid()
        base = wid * CH_PER_W

        pltpu.sync_copy(idx_hbm.at[pl.ds(base * GCHUNK, CH_PER_W * GCHUNK)],
                        idx_v)

        def gather(i, b):
            pltpu.async_copy(
                h_hbm.at[idx_v.at[pl.ds(i * GCHUNK, GCHUNK)]], bufs[b],
                sems[b])

        def drain(i, b):
            pltpu.make_async_copy(
                h_hbm.at[idx_v.at[pl.ds(i * GCHUNK, GCHUNK)]], bufs[b],
                sems[b]).wait()
            pltpu.sync_copy(bufs[b],
                            out_hbm.at[pl.ds((base + i) * GCHUNK, GCHUNK)])

        for b in range(NBUF):
            gather(b, b)

        def outer(o, _):
            for b in range(NBUF):
                i = o * NBUF + b
                drain(i, b)
                gather(i + NBUF, b)
            return _

        lax.fori_loop(0, OUTER - 1, outer, None)
        for b in range(NBUF):
            i = (OUTER - 1) * NBUF + b
            drain(i, b)
            if b < TAIL:
                gather(i + NBUF, b)
        for b in range(TAIL):
            i = OUTER * NBUF + b
            drain(i, b)

    return k(nbr_idx_flat, h)


def _medoid_aggregate(hn, dp, b):
    R = 8

    def body(hn_ref, dp_ref, b_ref, o_ref):
        dpb = dp_ref[...]
        w = jnp.where(dpb > 0.0, lax.rsqrt(jnp.maximum(dpb, 1e-30)), 0.0)
        rs = jnp.sum(w, axis=1, keepdims=True)
        ii = lax.broadcasted_iota(jnp.int32, (CAP, CAP), 0)
        jj = lax.broadcasted_iota(jnp.int32, (CAP, CAP), 1)
        eye = jnp.where(ii == jj, 1.0, 0.0)
        wT = lax.dot_general(eye, w, (((1,), (1,)), ((), ())),
                             preferred_element_type=jnp.float32)
        rows = []
        for r in range(R):
            hr = hn_ref[r]
            g = lax.dot_general(hr, hr, (((1,), (1,)), ((), ())),
                                preferred_element_type=jnp.float32)
            gd = g * eye
            sq_col = jnp.sum(gd, axis=1, keepdims=True)
            sq_row = jnp.sum(gd, axis=0, keepdims=True)
            d2 = jnp.maximum(sq_col + sq_row - 2.0 * g, 0.0)
            dist = jnp.sqrt(d2 + 1e-12)
            wcol = wT[:, r:r + 1]
            dk = jnp.sum(dist * wcol, axis=0, keepdims=True)
            valid = dpb[r:r + 1, :] > 0.0
            z = -dk / rs[r:r + 1, :]
            e = jnp.where(valid, jnp.exp(z), 0.0)
            uw = e * w[r:r + 1, :]
            s = jnp.sum(uw, axis=1, keepdims=True)
            wgt = uw / s
            o = lax.dot_general(wgt, hr, (((1,), (0,)), ((), ())),
                                preferred_element_type=jnp.float32)
            rows.append(rs[r:r + 1, :] * o)
        ob = jnp.concatenate(rows, axis=0) + b_ref[...]
        o_ref[...] = jnp.maximum(ob, 0.0)

    return pl.pallas_call(
        body,
        grid=(N // R,),
        in_specs=[
            pl.BlockSpec((R, CAP, D), lambda i: (i, 0, 0)),
            pl.BlockSpec((R, CAP), lambda i: (i, 0)),
            pl.BlockSpec((1, D), lambda i: (0, 0)),
        ],
        out_specs=pl.BlockSpec((R, D), lambda i: (i, 0)),
        out_shape=jax.ShapeDtypeStruct((N, D), jnp.float32),
        name="tc_medoid_aggregate",
    )(hn, dp, b)


def kernel(x, edge_index, W1, b1, W2, b2):
    pad = jnp.full((E_PAD - E,), SENT, jnp.int32)
    rows_pad = jnp.concatenate([edge_index[0].astype(jnp.int32), pad])
    cols_pad = jnp.concatenate([edge_index[1].astype(jnp.int32), pad])

    degp = _deg_partial(cols_pad)
    deg = _deg_merge(degp.reshape(NW, N_PAD // 128, 128))
    deg_flat = deg.reshape(N_PAD)

    nbr_idx, nbr_dp = _build_lists(rows_pad, cols_pad, deg_flat)
    dp2d = nbr_dp.reshape(N_PAD, CAP)
    idx_gather = nbr_idx[:FLAT_PAD]

    b1r = b1.reshape(1, D)
    b2r = b2.reshape(1, D)

    h1 = _matmul(x, W1)
    hn1 = _gather_rows(idx_gather, h1).reshape(FLAT_PAD // CAP, CAP, D)
    o1 = _medoid_aggregate(hn1, dp2d, b1r)

    h2 = _matmul(o1, W2)
    hn2 = _gather_rows(idx_gather, h2).reshape(FLAT_PAD // CAP, CAP, D)
    o2 = _medoid_aggregate(hn2, dp2d, b2r)
    return o2


# speedup vs baseline: 5.5362x; 5.5362x over previous
import functools

import jax
import jax.numpy as jnp
from jax import lax
from jax.experimental import pallas as pl
from jax.experimental.pallas import tpu as pltpu
from jax.experimental.pallas import tpu_sc as plsc

N = 10000
E = 160000
D = 128
CAP = 64

NC = 2
NS = 16
NW = NC * NS

N_PAD = 10240
ROWS_PER_W = N_PAD // NW
E_PAD = 160256
E_PER_W = E_PAD // NW
SENT = 1 << 20

FLAT = N * CAP
GCHUNK = 128
CH_PER_W = 157
NCHUNKS = NW * CH_PER_W
FLAT_PAD = NCHUNKS * GCHUNK
NBUF = 4

_SC_PARAMS = pltpu.CompilerParams(needs_layout_passes=False)


def _mesh():
    return plsc.VectorSubcoreMesh(core_axis_name="c", subcore_axis_name="s")


def _wid():
    return lax.axis_index("s") * NC + lax.axis_index("c")


def _deg_partial(cols_pad):
    @functools.partial(
        pl.kernel,
        mesh=_mesh(),
        out_type=jax.ShapeDtypeStruct((NW, N_PAD), jnp.float32),
        scratch_types=[
            pltpu.VMEM((N_PAD,), jnp.float32),
            pltpu.VMEM((E_PER_W,), jnp.int32),
        ],
        compiler_params=_SC_PARAMS,
        name="sc_deg_partial",
    )
    def k(cols_hbm, degp_hbm, hist_v, cbuf_v):
        wid = _wid()
        zeros16 = jnp.zeros((16,), jnp.float32)
        ones16 = jnp.ones((16,), jnp.float32)

        def zero_body(i, _):
            hist_v[pl.ds(i * 16, 16)] = zeros16
            return _

        lax.fori_loop(0, N_PAD // 16, zero_body, None)

        pltpu.sync_copy(cols_hbm.at[pl.ds(wid * E_PER_W, E_PER_W)], cbuf_v)

        def body(i, _):
            c = cbuf_v[pl.ds(i * 16, 16)]
            m = c < N
            plsc.addupdate_scatter(hist_v, [c], ones16, mask=m)
            return _

        lax.fori_loop(0, E_PER_W // 16, body, None)
        pltpu.sync_copy(hist_v, degp_hbm.at[wid])

    return k(cols_pad)


def _deg_merge(degp):
    def body(degp_ref, deg_ref):
        deg_ref[...] = jnp.sum(degp_ref[...], axis=0) + 1.0

    return pl.pallas_call(
        body,
        out_shape=jax.ShapeDtypeStruct((N_PAD // 128, 128), jnp.float32),
        name="tc_deg_merge",
    )(degp)


def _matmul(x, w):
    n = x.shape[0]
    blk = 400
    assert n % blk == 0

    def body(x_ref, w_ref, o_ref):
        o_ref[...] = lax.dot_general(
            x_ref[...], w_ref[...], (((1,), (0,)), ((), ())),
            preferred_element_type=jnp.float32)

    return pl.pallas_call(
        body,
        grid=(n // blk,),
        in_specs=[
            pl.BlockSpec((blk, D), lambda i: (i, 0)),
            pl.BlockSpec((D, D), lambda i: (0, 0)),
        ],
        out_specs=pl.BlockSpec((blk, D), lambda i: (i, 0)),
        out_shape=jax.ShapeDtypeStruct((n, D), jnp.float32),
        name="tc_matmul",
    )(x, w)


def _build_lists(rows_pad, cols_pad, deg_flat):
    ECHUNK = 2048
    NEC = E_PAD // ECHUNK if E_PAD % ECHUNK == 0 else E_PAD // ECHUNK + 1
    NFULL = E_PAD // ECHUNK
    TAIL = E_PAD - NFULL * ECHUNK

    @functools.partial(
        pl.kernel,
        mesh=_mesh(),
        out_type=[
            jax.ShapeDtypeStruct((N_PAD * CAP,), jnp.int32),
            jax.ShapeDtypeStruct((N_PAD * CAP,), jnp.float32),
        ],
        scratch_types=[
            pltpu.VMEM((N_PAD,), jnp.float32),
            pltpu.VMEM((ROWS_PER_W * CAP,), jnp.int32),
            pltpu.VMEM((ROWS_PER_W * CAP,), jnp.float32),
            pltpu.VMEM((ROWS_PER_W,), jnp.int32),
            pltpu.VMEM((ECHUNK,), jnp.int32),
            pltpu.VMEM((ECHUNK,), jnp.int32),
        ],
        compiler_params=_SC_PARAMS,
        name="sc_build_lists",
    )
    def k(rows_hbm, cols_hbm, deg_hbm, idx_hbm, dp_hbm,
          deg_v, idx_b, dp_b, cnt_v, rbuf, cbuf):
        wid = _wid()
        rlo = wid * ROWS_PER_W

        pltpu.sync_copy(deg_hbm, deg_v)

        zero16i = jnp.zeros((16,), jnp.int32)
        zero16f = jnp.zeros((16,), jnp.float32)
        one16i = jnp.ones((16,), jnp.int32)
        iota16 = lax.iota(jnp.int32, 16)

        def zb(i, _):
            pad16 = jnp.mod(rlo * CAP + i * 16 + iota16, N)
            idx_b[pl.ds(i * 16, 16)] = pad16
            dp_b[pl.ds(i * 16, 16)] = zero16f
            return _

        lax.fori_loop(0, ROWS_PER_W * CAP // 16, zb, None)

        def init_body(i, _):
            rl = iota16 + i * 16
            rg = rl + rlo
            ok = rg < N
            d = plsc.load_gather(deg_v, [rg], mask=ok)
            plsc.store_scatter(idx_b, [rl * CAP], rg, mask=ok)
            plsc.store_scatter(dp_b, [rl * CAP], d * d, mask=ok)
            cnt_v[pl.ds(i * 16, 16)] = one16i
            return _

        lax.fori_loop(0, ROWS_PER_W // 16, init_body, None)

        def process(nvec):
            def body(j, _):
                r = rbuf[pl.ds(j * 16, 16)]
                c = cbuf[pl.ds(j * 16, 16)]
                m = (r >= rlo) & (r < rlo + ROWS_PER_W)
                rl = jnp.where(m, r - rlo, ROWS_PER_W + iota16)
                occ, lastm = plsc.scan_count(rl, mask=m)
                base = plsc.load_gather(cnt_v, [rl], mask=m)
                slot = base + occ - 1
                ok = m & (slot < CAP)
                flat = jnp.where(ok, rl * CAP + slot, 0)
                plsc.store_scatter(idx_b, [flat], c, mask=ok)
                dr = plsc.load_gather(deg_v, [r], mask=m)
                dc = plsc.load_gather(deg_v, [c], mask=m)
                plsc.store_scatter(dp_b, [flat], dr * dc, mask=ok)
                newc = jnp.minimum(base + occ, CAP)
                plsc.store_scatter(cnt_v, [rl], newc, mask=m & lastm)
                return _

            lax.fori_loop(0, nvec, body, None)

        def chunk_body(ci, _):
            off = ci * ECHUNK
            pltpu.sync_copy(rows_hbm.at[pl.ds(off, ECHUNK)], rbuf)
            pltpu.sync_copy(cols_hbm.at[pl.ds(off, ECHUNK)], cbuf)
            process(ECHUNK // 16)
            return _

        lax.fori_loop(0, NFULL, chunk_body, None)
        if TAIL:
            off = NFULL * ECHUNK
            pltpu.sync_copy(rows_hbm.at[pl.ds(off, TAIL)],
                            rbuf.at[pl.ds(0, TAIL)])
            pltpu.sync_copy(cols_hbm.at[pl.ds(off, TAIL)],
                            cbuf.at[pl.ds(0, TAIL)])
            process(TAIL // 16)

        pltpu.sync_copy(idx_b, idx_hbm.at[pl.ds(rlo * CAP, ROWS_PER_W * CAP)])
        pltpu.sync_copy(dp_b, dp_hbm.at[pl.ds(rlo * CAP, ROWS_PER_W * CAP)])

    return k(rows_pad, cols_pad, deg_flat)


def _gather_rows(nbr_idx_flat, h):
    OUTER = CH_PER_W // NBUF
    TAIL = CH_PER_W - OUTER * NBUF

    @functools.partial(
        pl.kernel,
        mesh=_mesh(),
        out_type=jax.ShapeDtypeStruct((FLAT_PAD, D), jnp.float32),
        scratch_types=[
            pltpu.VMEM((CH_PER_W * GCHUNK,), jnp.int32),
            [pltpu.VMEM((GCHUNK, D), jnp.float32) for _ in range(NBUF)],
            [pltpu.SemaphoreType.DMA for _ in range(NBUF)],
        ],
        compiler_params=_SC_PARAMS,
        name="sc_gather_rows",
    )
    def k(idx_hbm, h_hbm, out_hbm, idx_v, bufs, sems):
        wid = _wid()
        base = wid * CH_PER_W

        pltpu.sync_copy(idx_hbm.at[pl.ds(base * GCHUNK, CH_PER_W * GCHUNK)],
                        idx_v)

        def gather(i, b):
            pltpu.async_copy(
                h_hbm.at[idx_v.at[pl.ds(i * GCHUNK, GCHUNK)]], bufs[b],
                sems[b])

        def drain(i, b):
            pltpu.make_async_copy(
                h_hbm.at[idx_v.at[pl.ds(i * GCHUNK, GCHUNK)]], bufs[b],
                sems[b]).wait()
            pltpu.sync_copy(bufs[b],
                            out_hbm.at[pl.ds((base + i) * GCHUNK, GCHUNK)])

        for b in range(NBUF):
            gather(b, b)

        def outer(o, _):
            for b in range(NBUF):
                i = o * NBUF + b
                drain(i, b)
                gather(i + NBUF, b)
            return _

        lax.fori_loop(0, OUTER - 1, outer, None)
        for b in range(NBUF):
            i = (OUTER - 1) * NBUF + b
            drain(i, b)
            if b < TAIL:
                gather(i + NBUF, b)
        for b in range(TAIL):
            i = OUTER * NBUF + b
            drain(i, b)

    return k(nbr_idx_flat, h)


def _medoid_aggregate(hn, dp, b):
    R = 8

    def body(hn_ref, dp_ref, b_ref, o_ref):
        dpb = dp_ref[...]
        w = jnp.where(dpb > 0.0, lax.rsqrt(jnp.maximum(dpb, 1e-30)), 0.0)
        rs = jnp.sum(w, axis=1, keepdims=True)
        ii = lax.broadcasted_iota(jnp.int32, (CAP, CAP), 0)
        jj = lax.broadcasted_iota(jnp.int32, (CAP, CAP), 1)
        eye = jnp.where(ii == jj, 1.0, 0.0)
        wT = lax.dot_general(eye, w, (((1,), (1,)), ((), ())),
                             preferred_element_type=jnp.float32)
        rows = []
        for r in range(R):
            hr = hn_ref[r]
            g = lax.dot_general(hr, hr, (((1,), (1,)), ((), ())),
                                preferred_element_type=jnp.float32)
            gd = g * eye
            sq_col = jnp.sum(gd, axis=1, keepdims=True)
            sq_row = jnp.sum(gd, axis=0, keepdims=True)
            d2 = jnp.maximum(sq_col + sq_row - 2.0 * g, 0.0)
            dist = jnp.sqrt(d2 + 1e-12)
            wcol = wT[:, r:r + 1]
            dk = jnp.sum(dist * wcol, axis=0, keepdims=True)
            valid = dpb[r:r + 1, :] > 0.0
            z = -dk / rs[r:r + 1, :]
            e = jnp.where(valid, jnp.exp(z), 0.0)
            uw = e * w[r:r + 1, :]
            s = jnp.sum(uw, axis=1, keepdims=True)
            wgt = uw / s
            o = lax.dot_general(wgt, hr, (((1,), (0,)), ((), ())),
                                preferred_element_type=jnp.float32)
            rows.append(rs[r:r + 1, :] * o)
        ob = jnp.concatenate(rows, axis=0) + b_ref[...]
        o_ref[...] = jnp.maximum(ob, 0.0)

    return pl.pallas_call(
        body,
        grid=(N // R,),
        in_specs=[
            pl.BlockSpec((R, CAP, D), lambda i: (i, 0, 0)),
            pl.BlockSpec((R, CAP), lambda i: (i, 0)),
            pl.BlockSpec((1, D), lambda i: (0, 0)),
        ],
        out_specs=pl.BlockSpec((R, D), lambda i: (i, 0)),
        out_shape=jax.ShapeDtypeStruct((N, D), jnp.float32),
        name="tc_medoid_aggregate",
    )(hn, dp, b)


def kernel(x, edge_index, W1, b1, W2, b2):
    pad = jnp.full((E_PAD - E,), SENT, jnp.int32)
    rows_pad = jnp.concatenate([edge_index[0].astype(jnp.int32), pad])
    cols_pad = jnp.concatenate([edge_index[1].astype(jnp.int32), pad])

    degp = _deg_partial(cols_pad)
    deg = _deg_merge(degp.reshape(NW, N_PAD // 128, 128))
    deg_flat = deg.reshape(N_PAD)

    nbr_idx, nbr_dp = _build_lists(rows_pad, cols_pad, deg_flat)
    dp2d = nbr_dp.reshape(N_PAD, CAP)
    idx_gather = nbr_idx[:FLAT_PAD]

    b1r = b1.reshape(1, D)
    b2r = b2.reshape(1, D)

    h1 = _matmul(x, W1)
    hn1 = _gather_rows(idx_gather, h1).reshape(FLAT_PAD // CAP, CAP, D)
    o1 = _medoid_aggregate(hn1, dp2d, b1r)

    h2 = _matmul(o1, W2)
    hn2 = _gather_rows(idx_gather, h2).reshape(FLAT_PAD // CAP, CAP, D)
    o2 = _medoid_aggregate(hn2, dp2d, b2r)
    return o2


# speedup vs baseline: 11.7110x; 2.1153x over previous
import functools

import jax
import jax.numpy as jnp
from jax import lax
from jax.experimental import pallas as pl
from jax.experimental.pallas import tpu as pltpu
from jax.experimental.pallas import tpu_sc as plsc

N = 10000
E = 160000
D = 128
CAP = 64

NC = 2
NS = 16
NW = NC * NS

N_PAD = 10240
ROWS_PER_W = N_PAD // NW
E_PAD = 160256
E_PER_W = E_PAD // NW
SENT = 1 << 20

FLAT = N * CAP
GCHUNK = 128
CH_PER_W = 157
NCHUNKS = NW * CH_PER_W
FLAT_PAD = NCHUNKS * GCHUNK
NBUF = 4

_SC_PARAMS = pltpu.CompilerParams(needs_layout_passes=False)


def _mesh():
    return plsc.VectorSubcoreMesh(core_axis_name="c", subcore_axis_name="s")


def _wid():
    return lax.axis_index("s") * NC + lax.axis_index("c")


def _deg_partial(cols_pad):
    @functools.partial(
        pl.kernel,
        mesh=_mesh(),
        out_type=jax.ShapeDtypeStruct((NW, N_PAD), jnp.float32),
        scratch_types=[
            pltpu.VMEM((N_PAD,), jnp.float32),
            pltpu.VMEM((E_PER_W,), jnp.int32),
        ],
        compiler_params=_SC_PARAMS,
        name="sc_deg_partial",
    )
    def k(cols_hbm, degp_hbm, hist_v, cbuf_v):
        wid = _wid()
        zeros16 = jnp.zeros((16,), jnp.float32)
        ones16 = jnp.ones((16,), jnp.float32)

        def zero_body(i, _):
            hist_v[pl.ds(i * 16, 16)] = zeros16
            return _

        lax.fori_loop(0, N_PAD // 16, zero_body, None)

        pltpu.sync_copy(cols_hbm.at[pl.ds(wid * E_PER_W, E_PER_W)], cbuf_v)

        def body(i, _):
            c = cbuf_v[pl.ds(i * 16, 16)]
            m = c < N
            plsc.addupdate_scatter(hist_v, [c], ones16, mask=m)
            return _

        lax.fori_loop(0, E_PER_W // 16, body, None)
        pltpu.sync_copy(hist_v, degp_hbm.at[wid])

    return k(cols_pad)


def _deg_merge(degp):
    def body(degp_ref, deg_ref):
        deg_ref[...] = jnp.sum(degp_ref[...], axis=0) + 1.0

    return pl.pallas_call(
        body,
        out_shape=jax.ShapeDtypeStruct((N_PAD // 128, 128), jnp.float32),
        name="tc_deg_merge",
    )(degp)


def _matmul(x, w):
    n = x.shape[0]
    blk = 400
    assert n % blk == 0

    def body(x_ref, w_ref, o_ref):
        o_ref[...] = lax.dot_general(
            x_ref[...], w_ref[...], (((1,), (0,)), ((), ())),
            preferred_element_type=jnp.float32)

    return pl.pallas_call(
        body,
        grid=(n // blk,),
        in_specs=[
            pl.BlockSpec((blk, D), lambda i: (i, 0)),
            pl.BlockSpec((D, D), lambda i: (0, 0)),
        ],
        out_specs=pl.BlockSpec((blk, D), lambda i: (i, 0)),
        out_shape=jax.ShapeDtypeStruct((n, D), jnp.float32),
        name="tc_matmul",
    )(x, w)


def _build_lists(rows_pad, cols_pad, deg_flat):
    ECHUNK = 2048
    NEC = E_PAD // ECHUNK if E_PAD % ECHUNK == 0 else E_PAD // ECHUNK + 1
    NFULL = E_PAD // ECHUNK
    TAIL = E_PAD - NFULL * ECHUNK

    @functools.partial(
        pl.kernel,
        mesh=_mesh(),
        out_type=[
            jax.ShapeDtypeStruct((N_PAD * CAP,), jnp.int32),
            jax.ShapeDtypeStruct((N_PAD * CAP,), jnp.float32),
        ],
        scratch_types=[
            pltpu.VMEM((N_PAD,), jnp.float32),
            pltpu.VMEM((ROWS_PER_W * CAP,), jnp.int32),
            pltpu.VMEM((ROWS_PER_W * CAP,), jnp.float32),
            pltpu.VMEM((ROWS_PER_W,), jnp.int32),
            pltpu.VMEM((ECHUNK,), jnp.int32),
            pltpu.VMEM((ECHUNK,), jnp.int32),
        ],
        compiler_params=_SC_PARAMS,
        name="sc_build_lists",
    )
    def k(rows_hbm, cols_hbm, deg_hbm, idx_hbm, dp_hbm,
          deg_v, idx_b, dp_b, cnt_v, rbuf, cbuf):
        wid = _wid()
        rlo = wid * ROWS_PER_W

        pltpu.sync_copy(deg_hbm, deg_v)

        zero16i = jnp.zeros((16,), jnp.int32)
        zero16f = jnp.zeros((16,), jnp.float32)
        one16i = jnp.ones((16,), jnp.int32)
        iota16 = lax.iota(jnp.int32, 16)

        def zb(i, _):
            pad16 = jnp.mod(rlo * CAP + i * 16 + iota16, N)
            idx_b[pl.ds(i * 16, 16)] = pad16
            dp_b[pl.ds(i * 16, 16)] = zero16f
            return _

        lax.fori_loop(0, ROWS_PER_W * CAP // 16, zb, None)

        def init_body(i, _):
            rl = iota16 + i * 16
            rg = rl + rlo
            ok = rg < N
            d = plsc.load_gather(deg_v, [rg], mask=ok)
            plsc.store_scatter(idx_b, [rl * CAP], rg, mask=ok)
            plsc.store_scatter(dp_b, [rl * CAP], d * d, mask=ok)
            cnt_v[pl.ds(i * 16, 16)] = one16i
            return _

        lax.fori_loop(0, ROWS_PER_W // 16, init_body, None)

        def process(nvec):
            def body(j, _):
                r = rbuf[pl.ds(j * 16, 16)]
                c = cbuf[pl.ds(j * 16, 16)]
                m = (r >= rlo) & (r < rlo + ROWS_PER_W)
                rl = jnp.where(m, r - rlo, ROWS_PER_W + iota16)
                occ, lastm = plsc.scan_count(rl, mask=m)
                base = plsc.load_gather(cnt_v, [rl], mask=m)
                slot = base + occ - 1
                ok = m & (slot < CAP)
                flat = jnp.where(ok, rl * CAP + slot, 0)
                plsc.store_scatter(idx_b, [flat], c, mask=ok)
                dr = plsc.load_gather(deg_v, [r], mask=m)
                dc = plsc.load_gather(deg_v, [c], mask=m)
                plsc.store_scatter(dp_b, [flat], dr * dc, mask=ok)
                newc = jnp.minimum(base + occ, CAP)
                plsc.store_scatter(cnt_v, [rl], newc, mask=m & lastm)
                return _

            lax.fori_loop(0, nvec, body, None)

        def chunk_body(ci, _):
            off = ci * ECHUNK
            pltpu.sync_copy(rows_hbm.at[pl.ds(off, ECHUNK)], rbuf)
            pltpu.sync_copy(cols_hbm.at[pl.ds(off, ECHUNK)], cbuf)
            process(ECHUNK // 16)
            return _

        lax.fori_loop(0, NFULL, chunk_body, None)
        if TAIL:
            off = NFULL * ECHUNK
            pltpu.sync_copy(rows_hbm.at[pl.ds(off, TAIL)],
                            rbuf.at[pl.ds(0, TAIL)])
            pltpu.sync_copy(cols_hbm.at[pl.ds(off, TAIL)],
                            cbuf.at[pl.ds(0, TAIL)])
            process(TAIL // 16)

        pltpu.sync_copy(idx_b, idx_hbm.at[pl.ds(rlo * CAP, ROWS_PER_W * CAP)])
        pltpu.sync_copy(dp_b, dp_hbm.at[pl.ds(rlo * CAP, ROWS_PER_W * CAP)])

    return k(rows_pad, cols_pad, deg_flat)


def _gather_rows(nbr_idx_flat, h):
    OUTER = CH_PER_W // NBUF
    TAIL = CH_PER_W - OUTER * NBUF

    @functools.partial(
        pl.kernel,
        mesh=_mesh(),
        out_type=jax.ShapeDtypeStruct((FLAT_PAD, D), jnp.float32),
        scratch_types=[
            pltpu.VMEM((CH_PER_W * GCHUNK,), jnp.int32),
            [pltpu.VMEM((GCHUNK, D), jnp.float32) for _ in range(NBUF)],
            [pltpu.SemaphoreType.DMA for _ in range(NBUF)],
        ],
        compiler_params=_SC_PARAMS,
        name="sc_gather_rows",
    )
    def k(idx_hbm, h_hbm, out_hbm, idx_v, bufs, sems):
        wid = _wid()
        base = wid * CH_PER_W

        pltpu.sync_copy(idx_hbm.at[pl.ds(base * GCHUNK, CH_PER_W * GCHUNK)],
                        idx_v)

        def gather(i, b):
            pltpu.async_copy(
                h_hbm.at[idx_v.at[pl.ds(i * GCHUNK, GCHUNK)]], bufs[b],
                sems[b])

        def drain(i, b):
            pltpu.make_async_copy(
                h_hbm.at[idx_v.at[pl.ds(i * GCHUNK, GCHUNK)]], bufs[b],
                sems[b]).wait()
            pltpu.sync_copy(bufs[b],
                            out_hbm.at[pl.ds((base + i) * GCHUNK, GCHUNK)])

        for b in range(NBUF):
            gather(b, b)

        def outer(o, _):
            for b in range(NBUF):
                i = o * NBUF + b
                drain(i, b)
                gather(i + NBUF, b)
            return _

        lax.fori_loop(0, OUTER - 1, outer, None)
        for b in range(NBUF):
            i = (OUTER - 1) * NBUF + b
            drain(i, b)
            if b < TAIL:
                gather(i + NBUF, b)
        for b in range(TAIL):
            i = OUTER * NBUF + b
            drain(i, b)

    return k(nbr_idx_flat, h)


def _medoid_aggregate(hn, dp, b):
    R = 8

    RC = R * CAP

    def body(hn_ref, dp_ref, b_ref, o_ref):
        f32 = jnp.float32

        def dot(a, bb, dims):
            return lax.dot_general(a, bb, (dims, ((), ())),
                                   preferred_element_type=f32)

        dpb = dp_ref[...]
        w = jnp.where(dpb > 0.0, lax.rsqrt(jnp.maximum(dpb, 1e-30)), 0.0)
        rs = jnp.sum(w, axis=1, keepdims=True)

        rr = lax.broadcasted_iota(jnp.int32, (R, RC), 0)
        aa = lax.broadcasted_iota(jnp.int32, (R, RC), 1)
        sel = jnp.where(aa // CAP == rr, 1.0, 0.0)
        a2 = lax.broadcasted_iota(jnp.int32, (RC, CAP), 0)
        j2 = lax.broadcasted_iota(jnp.int32, (RC, CAP), 1)
        eyeT = jnp.where(a2 % CAP == j2, 1.0, 0.0)

        gs = []
        for r in range(R):
            hr = hn_ref[r]
            gs.append(dot(hr, hr, ((1,), (1,))))
        g_all = jnp.concatenate(gs, axis=0)

        gd = g_all * eyeT
        sq_col = jnp.sum(gd, axis=1, keepdims=True)
        sq_rows = dot(sel, gd, ((1,), (0,)))
        sq_exp = dot(sel, sq_rows, ((0,), (0,)))
        d2 = jnp.maximum(sq_col + sq_exp - 2.0 * g_all, 0.0)
        dist = jnp.sqrt(d2 + 1e-12)

        wx = dot(sel, w, ((0,), (0,)))
        w_col = jnp.sum(wx * eyeT, axis=1, keepdims=True)
        dk = dot(sel, dist * w_col, ((1,), (0,)))

        valid = dpb > 0.0
        z = -dk / rs
        e = jnp.where(valid, jnp.exp(z), 0.0)
        uw = e * w
        s = jnp.sum(uw, axis=1, keepdims=True)
        wgt = uw / s

        wg_exp = dot(sel, wgt, ((0,), (0,)))
        w2_col = jnp.sum(wg_exp * eyeT, axis=1, keepdims=True)
        hc = hn_ref[...].reshape(RC, D)
        out = dot(sel, hc * w2_col, ((1,), (0,)))
        ob = rs * out + b_ref[...]
        o_ref[...] = jnp.maximum(ob, 0.0)

    return pl.pallas_call(
        body,
        grid=(N // R,),
        in_specs=[
            pl.BlockSpec((R, CAP, D), lambda i: (i, 0, 0)),
            pl.BlockSpec((R, CAP), lambda i: (i, 0)),
            pl.BlockSpec((1, D), lambda i: (0, 0)),
        ],
        out_specs=pl.BlockSpec((R, D), lambda i: (i, 0)),
        out_shape=jax.ShapeDtypeStruct((N, D), jnp.float32),
        name="tc_medoid_aggregate",
    )(hn, dp, b)


def kernel(x, edge_index, W1, b1, W2, b2):
    pad = jnp.full((E_PAD - E,), SENT, jnp.int32)
    rows_pad = jnp.concatenate([edge_index[0].astype(jnp.int32), pad])
    cols_pad = jnp.concatenate([edge_index[1].astype(jnp.int32), pad])

    degp = _deg_partial(cols_pad)
    deg = _deg_merge(degp.reshape(NW, N_PAD // 128, 128))
    deg_flat = deg.reshape(N_PAD)

    nbr_idx, nbr_dp = _build_lists(rows_pad, cols_pad, deg_flat)
    dp2d = nbr_dp.reshape(N_PAD, CAP)
    idx_gather = nbr_idx[:FLAT_PAD]

    b1r = b1.reshape(1, D)
    b2r = b2.reshape(1, D)

    h1 = _matmul(x, W1)
    hn1 = _gather_rows(idx_gather, h1).reshape(FLAT_PAD // CAP, CAP, D)
    o1 = _medoid_aggregate(hn1, dp2d, b1r)

    h2 = _matmul(o1, W2)
    hn2 = _gather_rows(idx_gather, h2).reshape(FLAT_PAD // CAP, CAP, D)
    o2 = _medoid_aggregate(hn2, dp2d, b2r)
    return o2


# speedup vs baseline: 15.0073x; 1.2815x over previous
import functools

import jax
import jax.numpy as jnp
from jax import lax
from jax.experimental import pallas as pl
from jax.experimental.pallas import tpu as pltpu
from jax.experimental.pallas import tpu_sc as plsc

N = 10000
E = 160000
D = 128
CAP = 64

NC = 2
NS = 16
NW = NC * NS

N_PAD = 10240
ROWS_PER_W = N_PAD // NW
E_PAD = 160256
E_PER_W = E_PAD // NW
SENT = 1 << 20

FLAT = N * CAP
GCHUNK = 128
CH_PER_W = 157
NCHUNKS = NW * CH_PER_W
FLAT_PAD = NCHUNKS * GCHUNK
NBUF = 4

_SC_PARAMS = pltpu.CompilerParams(needs_layout_passes=False)


def _mesh():
    return plsc.VectorSubcoreMesh(core_axis_name="c", subcore_axis_name="s")


def _wid():
    return lax.axis_index("s") * NC + lax.axis_index("c")


def _deg_partial(cols_pad):
    @functools.partial(
        pl.kernel,
        mesh=_mesh(),
        out_type=jax.ShapeDtypeStruct((NW, N_PAD), jnp.float32),
        scratch_types=[
            pltpu.VMEM((N_PAD,), jnp.float32),
            pltpu.VMEM((E_PER_W,), jnp.int32),
        ],
        compiler_params=_SC_PARAMS,
        name="sc_deg_partial",
    )
    def k(cols_hbm, degp_hbm, hist_v, cbuf_v):
        wid = _wid()
        zeros16 = jnp.zeros((16,), jnp.float32)
        ones16 = jnp.ones((16,), jnp.float32)

        def zero_body(i, _):
            hist_v[pl.ds(i * 16, 16)] = zeros16
            return _

        lax.fori_loop(0, N_PAD // 16, zero_body, None)

        pltpu.sync_copy(cols_hbm.at[pl.ds(wid * E_PER_W, E_PER_W)], cbuf_v)

        def body(i, _):
            c = cbuf_v[pl.ds(i * 16, 16)]
            m = c < N
            plsc.addupdate_scatter(hist_v, [c], ones16, mask=m)
            return _

        lax.fori_loop(0, E_PER_W // 16, body, None)
        pltpu.sync_copy(hist_v, degp_hbm.at[wid])

    return k(cols_pad)


def _deg_merge(degp):
    def body(degp_ref, deg_ref):
        deg_ref[...] = jnp.sum(degp_ref[...], axis=0) + 1.0

    return pl.pallas_call(
        body,
        out_shape=jax.ShapeDtypeStruct((N_PAD // 128, 128), jnp.float32),
        name="tc_deg_merge",
    )(degp)


def _matmul(x, w):
    n = x.shape[0]
    blk = 400
    assert n % blk == 0

    def body(x_ref, w_ref, o_ref):
        o_ref[...] = lax.dot_general(
            x_ref[...], w_ref[...], (((1,), (0,)), ((), ())),
            preferred_element_type=jnp.float32)

    return pl.pallas_call(
        body,
        grid=(n // blk,),
        in_specs=[
            pl.BlockSpec((blk, D), lambda i: (i, 0)),
            pl.BlockSpec((D, D), lambda i: (0, 0)),
        ],
        out_specs=pl.BlockSpec((blk, D), lambda i: (i, 0)),
        out_shape=jax.ShapeDtypeStruct((n, D), jnp.float32),
        name="tc_matmul",
    )(x, w)


def _build_lists(rows_pad, cols_pad, deg_flat):
    ECHUNK = 2048
    NEC = E_PAD // ECHUNK if E_PAD % ECHUNK == 0 else E_PAD // ECHUNK + 1
    NFULL = E_PAD // ECHUNK
    TAIL = E_PAD - NFULL * ECHUNK

    @functools.partial(
        pl.kernel,
        mesh=_mesh(),
        out_type=[
            jax.ShapeDtypeStruct((N_PAD * CAP,), jnp.int32),
            jax.ShapeDtypeStruct((N_PAD * CAP,), jnp.float32),
        ],
        scratch_types=[
            pltpu.VMEM((N_PAD,), jnp.float32),
            pltpu.VMEM((ROWS_PER_W * CAP,), jnp.int32),
            pltpu.VMEM((ROWS_PER_W * CAP,), jnp.float32),
            pltpu.VMEM((ROWS_PER_W,), jnp.int32),
            pltpu.VMEM((ECHUNK,), jnp.int32),
            pltpu.VMEM((ECHUNK,), jnp.int32),
        ],
        compiler_params=_SC_PARAMS,
        name="sc_build_lists",
    )
    def k(rows_hbm, cols_hbm, deg_hbm, idx_hbm, dp_hbm,
          deg_v, idx_b, dp_b, cnt_v, rbuf, cbuf):
        wid = _wid()
        rlo = wid * ROWS_PER_W

        pltpu.sync_copy(deg_hbm, deg_v)

        zero16i = jnp.zeros((16,), jnp.int32)
        zero16f = jnp.zeros((16,), jnp.float32)
        one16i = jnp.ones((16,), jnp.int32)
        iota16 = lax.iota(jnp.int32, 16)

        def zb(i, _):
            pad16 = jnp.mod(rlo * CAP + i * 16 + iota16, N)
            idx_b[pl.ds(i * 16, 16)] = pad16
            dp_b[pl.ds(i * 16, 16)] = zero16f
            return _

        lax.fori_loop(0, ROWS_PER_W * CAP // 16, zb, None)

        def init_body(i, _):
            rl = iota16 + i * 16
            rg = rl + rlo
            ok = rg < N
            d = plsc.load_gather(deg_v, [rg], mask=ok)
            plsc.store_scatter(idx_b, [rl * CAP], rg, mask=ok)
            plsc.store_scatter(dp_b, [rl * CAP], d * d, mask=ok)
            cnt_v[pl.ds(i * 16, 16)] = one16i
            return _

        lax.fori_loop(0, ROWS_PER_W // 16, init_body, None)

        def process(nvec):
            def body(j, _):
                r = rbuf[pl.ds(j * 16, 16)]
                c = cbuf[pl.ds(j * 16, 16)]
                m = (r >= rlo) & (r < rlo + ROWS_PER_W)
                rl = jnp.where(m, r - rlo, ROWS_PER_W + iota16)
                occ, lastm = plsc.scan_count(rl, mask=m)
                base = plsc.load_gather(cnt_v, [rl], mask=m)
                slot = base + occ - 1
                ok = m & (slot < CAP)
                flat = jnp.where(ok, rl * CAP + slot, 0)
                plsc.store_scatter(idx_b, [flat], c, mask=ok)
                dr = plsc.load_gather(deg_v, [r], mask=m)
                dc = plsc.load_gather(deg_v, [c], mask=m)
                plsc.store_scatter(dp_b, [flat], dr * dc, mask=ok)
                newc = jnp.minimum(base + occ, CAP)
                plsc.store_scatter(cnt_v, [rl], newc, mask=m & lastm)
                return _

            lax.fori_loop(0, nvec, body, None)

        def chunk_body(ci, _):
            off = ci * ECHUNK
            pltpu.sync_copy(rows_hbm.at[pl.ds(off, ECHUNK)], rbuf)
            pltpu.sync_copy(cols_hbm.at[pl.ds(off, ECHUNK)], cbuf)
            process(ECHUNK // 16)
            return _

        lax.fori_loop(0, NFULL, chunk_body, None)
        if TAIL:
            off = NFULL * ECHUNK
            pltpu.sync_copy(rows_hbm.at[pl.ds(off, TAIL)],
                            rbuf.at[pl.ds(0, TAIL)])
            pltpu.sync_copy(cols_hbm.at[pl.ds(off, TAIL)],
                            cbuf.at[pl.ds(0, TAIL)])
            process(TAIL // 16)

        pltpu.sync_copy(idx_b, idx_hbm.at[pl.ds(rlo * CAP, ROWS_PER_W * CAP)])
        pltpu.sync_copy(dp_b, dp_hbm.at[pl.ds(rlo * CAP, ROWS_PER_W * CAP)])

    return k(rows_pad, cols_pad, deg_flat)


def _gather_rows(nbr_idx_flat, h):
    OUTER = CH_PER_W // NBUF
    TAIL = CH_PER_W - OUTER * NBUF

    @functools.partial(
        pl.kernel,
        mesh=_mesh(),
        out_type=jax.ShapeDtypeStruct((FLAT_PAD, D), jnp.float32),
        scratch_types=[
            pltpu.VMEM((CH_PER_W * GCHUNK,), jnp.int32),
            [pltpu.VMEM((GCHUNK, D), jnp.float32) for _ in range(NBUF)],
            [pltpu.SemaphoreType.DMA for _ in range(NBUF)],
        ],
        compiler_params=_SC_PARAMS,
        name="sc_gather_rows",
    )
    def k(idx_hbm, h_hbm, out_hbm, idx_v, bufs, sems):
        wid = _wid()
        base = wid * CH_PER_W

        pltpu.sync_copy(idx_hbm.at[pl.ds(base * GCHUNK, CH_PER_W * GCHUNK)],
                        idx_v)

        def gather(i, b):
            pltpu.async_copy(
                h_hbm.at[idx_v.at[pl.ds(i * GCHUNK, GCHUNK)]], bufs[b],
                sems[b])

        def drain(i, b):
            pltpu.make_async_copy(
                h_hbm.at[idx_v.at[pl.ds(i * GCHUNK, GCHUNK)]], bufs[b],
                sems[b]).wait()
            pltpu.sync_copy(bufs[b],
                            out_hbm.at[pl.ds((base + i) * GCHUNK, GCHUNK)])

        for b in range(NBUF):
            gather(b, b)

        def outer(o, _):
            for b in range(NBUF):
                i = o * NBUF + b
                drain(i, b)
                gather(i + NBUF, b)
            return _

        lax.fori_loop(0, OUTER - 1, outer, None)
        for b in range(NBUF):
            i = (OUTER - 1) * NBUF + b
            drain(i, b)
            if b < TAIL:
                gather(i + NBUF, b)
        for b in range(TAIL):
            i = OUTER * NBUF + b
            drain(i, b)

    return k(nbr_idx_flat, h)


def _medoid_aggregate(hn, dp, b, sel, eyeT):
    R = 8

    RC = R * CAP

    def body(hn_ref, dp_ref, b_ref, sel_ref, eyeT_ref, o_ref):
        f32 = jnp.float32

        def dot(a, bb, dims):
            return lax.dot_general(a, bb, (dims, ((), ())),
                                   preferred_element_type=f32)

        dpb = dp_ref[...]
        w = jnp.where(dpb > 0.0, lax.rsqrt(jnp.maximum(dpb, 1e-30)), 0.0)
        rs = jnp.sum(w, axis=1, keepdims=True)

        sel = sel_ref[...]
        eyeT = eyeT_ref[...]

        gs = []
        for r in range(R):
            hr = hn_ref[r]
            gs.append(dot(hr, hr, ((1,), (1,))))
        g_all = jnp.concatenate(gs, axis=0)

        def bexp(x):
            return jnp.broadcast_to(x[:, None, :], (R, CAP, CAP)).reshape(RC, CAP)

        def bsum(x, n):
            return jnp.sum(x.reshape(R, CAP, n), axis=1)

        gd = g_all * eyeT
        sq_col = jnp.sum(gd, axis=1, keepdims=True)
        sq_rows = bsum(gd, CAP)
        d2 = jnp.maximum(sq_col + bexp(sq_rows) - 2.0 * g_all, 0.0)
        dist = jnp.sqrt(d2 + 1e-12)
        dist = jnp.where(eyeT > 0.5, 1e-6, dist)

        w_col = jnp.sum(bexp(w) * eyeT, axis=1, keepdims=True)
        dk = bsum(dist * w_col, CAP)

        valid = dpb > 0.0
        z = -dk / rs
        e = jnp.where(valid, jnp.exp(z), 0.0)
        uw = e * w
        s = jnp.sum(uw, axis=1, keepdims=True)
        wgt = uw / s

        w2_col = jnp.sum(bexp(wgt) * eyeT, axis=1, keepdims=True)
        hc = hn_ref[...].reshape(RC, D)
        out = bsum(hc * w2_col, D)
        ob = rs * out + b_ref[...]
        o_ref[...] = jnp.maximum(ob, 0.0)

    return pl.pallas_call(
        body,
        grid=(N // R,),
        in_specs=[
            pl.BlockSpec((R, CAP, D), lambda i: (i, 0, 0)),
            pl.BlockSpec((R, CAP), lambda i: (i, 0)),
            pl.BlockSpec((1, D), lambda i: (0, 0)),
            pl.BlockSpec((R, R * CAP), lambda i: (0, 0)),
            pl.BlockSpec((R * CAP, CAP), lambda i: (0, 0)),
        ],
        out_specs=pl.BlockSpec((R, D), lambda i: (i, 0)),
        out_shape=jax.ShapeDtypeStruct((N, D), jnp.float32),
        name="tc_medoid_aggregate",
    )(hn, dp, b, sel, eyeT)


def kernel(x, edge_index, W1, b1, W2, b2):
    pad = jnp.full((E_PAD - E,), SENT, jnp.int32)
    rows_pad = jnp.concatenate([edge_index[0].astype(jnp.int32), pad])
    cols_pad = jnp.concatenate([edge_index[1].astype(jnp.int32), pad])

    degp = _deg_partial(cols_pad)
    deg = _deg_merge(degp.reshape(NW, N_PAD // 128, 128))
    deg_flat = deg.reshape(N_PAD)

    nbr_idx, nbr_dp = _build_lists(rows_pad, cols_pad, deg_flat)
    dp2d = nbr_dp.reshape(N_PAD, CAP)
    idx_gather = nbr_idx[:FLAT_PAD]

    b1r = b1.reshape(1, D)
    b2r = b2.reshape(1, D)

    R = 8
    RC = R * CAP
    sel = (jnp.arange(RC, dtype=jnp.int32)[None, :] // CAP
           == jnp.arange(R, dtype=jnp.int32)[:, None]).astype(jnp.float32)
    eyeT = (jnp.arange(RC, dtype=jnp.int32)[:, None] % CAP
            == jnp.arange(CAP, dtype=jnp.int32)[None, :]).astype(jnp.float32)

    h1 = _matmul(x, W1)
    hn1 = _gather_rows(idx_gather, h1).reshape(FLAT_PAD // CAP, CAP, D)
    o1 = _medoid_aggregate(hn1, dp2d, b1r, sel, eyeT)

    h2 = _matmul(o1, W2)
    hn2 = _gather_rows(idx_gather, h2).reshape(FLAT_PAD // CAP, CAP, D)
    o2 = _medoid_aggregate(hn2, dp2d, b2r, sel, eyeT)
    return o2


# speedup vs baseline: 24.7571x; 1.6497x over previous
import functools

import jax
import jax.numpy as jnp
from jax import lax
from jax.experimental import pallas as pl
from jax.experimental.pallas import tpu as pltpu
from jax.experimental.pallas import tpu_sc as plsc

N = 10000
E = 160000
D = 128
CAP = 64

NC = 2
NS = 16
NW = NC * NS

N_PAD = 10240
ROWS_PER_W = N_PAD // NW
E_PAD = 160256
E_PER_W = E_PAD // NW
SENT = 1 << 20

FLAT = N * CAP
GCHUNK = 128
CH_PER_W = 157
NCHUNKS = NW * CH_PER_W
FLAT_PAD = NCHUNKS * GCHUNK
NBUF = 4

_SC_PARAMS = pltpu.CompilerParams(needs_layout_passes=False)


def _mesh():
    return plsc.VectorSubcoreMesh(core_axis_name="c", subcore_axis_name="s")


def _wid():
    return lax.axis_index("s") * NC + lax.axis_index("c")


def _deg_partial(cols_pad):
    @functools.partial(
        pl.kernel,
        mesh=_mesh(),
        out_type=jax.ShapeDtypeStruct((NW, N_PAD), jnp.float32),
        scratch_types=[
            pltpu.VMEM((N_PAD,), jnp.float32),
            pltpu.VMEM((E_PER_W,), jnp.int32),
        ],
        compiler_params=_SC_PARAMS,
        name="sc_deg_partial",
    )
    def k(cols_hbm, degp_hbm, hist_v, cbuf_v):
        wid = _wid()
        zeros16 = jnp.zeros((16,), jnp.float32)
        ones16 = jnp.ones((16,), jnp.float32)

        def zero_body(i, _):
            hist_v[pl.ds(i * 16, 16)] = zeros16
            return _

        lax.fori_loop(0, N_PAD // 16, zero_body, None)

        pltpu.sync_copy(cols_hbm.at[pl.ds(wid * E_PER_W, E_PER_W)], cbuf_v)

        def body(i, _):
            c = cbuf_v[pl.ds(i * 16, 16)]
            m = c < N
            plsc.addupdate_scatter(hist_v, [c], ones16, mask=m)
            return _

        lax.fori_loop(0, E_PER_W // 16, body, None)
        pltpu.sync_copy(hist_v, degp_hbm.at[wid])

    return k(cols_pad)


def _matmul_deg(x, w, degp):
    n = x.shape[0]
    blk = 400
    assert n % blk == 0

    def body(x_ref, w_ref, degp_ref, o_ref, deg_ref):
        o_ref[...] = lax.dot_general(
            x_ref[...], w_ref[...], (((1,), (0,)), ((), ())),
            preferred_element_type=jnp.float32)

        @pl.when(pl.program_id(0) == 0)
        def _():
            deg_ref[...] = jnp.sum(degp_ref[...], axis=0) + 1.0

    return pl.pallas_call(
        body,
        grid=(n // blk,),
        in_specs=[
            pl.BlockSpec((blk, D), lambda i: (i, 0)),
            pl.BlockSpec((D, D), lambda i: (0, 0)),
            pl.BlockSpec((NW, N_PAD // 128, 128), lambda i: (0, 0, 0)),
        ],
        out_specs=[
            pl.BlockSpec((blk, D), lambda i: (i, 0)),
            pl.BlockSpec((N_PAD // 128, 128), lambda i: (0, 0)),
        ],
        out_shape=[
            jax.ShapeDtypeStruct((n, D), jnp.float32),
            jax.ShapeDtypeStruct((N_PAD // 128, 128), jnp.float32),
        ],
        name="tc_matmul_deg",
    )(x, w, degp)


def _build_lists(rows_pad, cols_pad, deg_flat):
    ECHUNK = 2048
    NEC = E_PAD // ECHUNK if E_PAD % ECHUNK == 0 else E_PAD // ECHUNK + 1
    NFULL = E_PAD // ECHUNK
    TAIL = E_PAD - NFULL * ECHUNK

    @functools.partial(
        pl.kernel,
        mesh=_mesh(),
        out_type=[
            jax.ShapeDtypeStruct((N_PAD * CAP,), jnp.int32),
            jax.ShapeDtypeStruct((N_PAD * CAP,), jnp.float32),
        ],
        scratch_types=[
            pltpu.VMEM((N_PAD,), jnp.float32),
            pltpu.VMEM((ROWS_PER_W * CAP,), jnp.int32),
            pltpu.VMEM((ROWS_PER_W * CAP,), jnp.float32),
            pltpu.VMEM((ROWS_PER_W,), jnp.int32),
            pltpu.VMEM((ECHUNK,), jnp.int32),
            pltpu.VMEM((ECHUNK,), jnp.int32),
        ],
        compiler_params=_SC_PARAMS,
        name="sc_build_lists",
    )
    def k(rows_hbm, cols_hbm, deg_hbm, idx_hbm, dp_hbm,
          deg_v, idx_b, dp_b, cnt_v, rbuf, cbuf):
        wid = _wid()
        rlo = wid * ROWS_PER_W

        pltpu.sync_copy(deg_hbm, deg_v)

        zero16i = jnp.zeros((16,), jnp.int32)
        zero16f = jnp.zeros((16,), jnp.float32)
        one16i = jnp.ones((16,), jnp.int32)
        iota16 = lax.iota(jnp.int32, 16)

        def zb(i, _):
            pad16 = jnp.mod(rlo * CAP + i * 16 + iota16, N)
            idx_b[pl.ds(i * 16, 16)] = pad16
            return _

        lax.fori_loop(0, ROWS_PER_W * CAP // 16, zb, None)

        def init_body(i, _):
            rl = iota16 + i * 16
            rg = rl + rlo
            ok = rg < N
            plsc.store_scatter(idx_b, [rl * CAP], rg, mask=ok)
            cnt_v[pl.ds(i * 16, 16)] = one16i
            return _

        lax.fori_loop(0, ROWS_PER_W // 16, init_body, None)

        def process(nvec):
            def body(j, _):
                r = rbuf[pl.ds(j * 16, 16)]
                c = cbuf[pl.ds(j * 16, 16)]
                m = (r >= rlo) & (r < rlo + ROWS_PER_W)
                rl = jnp.where(m, r - rlo, ROWS_PER_W + iota16)
                occ, lastm = plsc.scan_count(rl, mask=m)
                base = plsc.load_gather(cnt_v, [rl], mask=m)
                slot = base + occ - 1
                ok = m & (slot < CAP)
                flat = jnp.where(ok, rl * CAP + slot, 0)
                plsc.store_scatter(idx_b, [flat], c, mask=ok)
                newc = jnp.minimum(base + occ, CAP)
                plsc.store_scatter(cnt_v, [rl], newc, mask=m & lastm)
                return _

            lax.fori_loop(0, nvec, body, None)

        def chunk_body(ci, _):
            off = ci * ECHUNK
            pltpu.sync_copy(rows_hbm.at[pl.ds(off, ECHUNK)], rbuf)
            pltpu.sync_copy(cols_hbm.at[pl.ds(off, ECHUNK)], cbuf)
            process(ECHUNK // 16)
            return _

        lax.fori_loop(0, NFULL, chunk_body, None)
        if TAIL:
            off = NFULL * ECHUNK
            pltpu.sync_copy(rows_hbm.at[pl.ds(off, TAIL)],
                            rbuf.at[pl.ds(0, TAIL)])
            pltpu.sync_copy(cols_hbm.at[pl.ds(off, TAIL)],
                            cbuf.at[pl.ds(0, TAIL)])
            process(TAIL // 16)

        def dp_outer(g, _):
            rl16 = iota16 + g * 16
            rg16 = rl16 + rlo
            dr16 = plsc.load_gather(deg_v, [rg16])
            cnt16 = plsc.load_gather(cnt_v, [rl16])
            for sl in range(CAP):
                flat16 = rl16 * CAP + sl
                idx16 = plsc.load_gather(idx_b, [flat16])
                d16 = plsc.load_gather(deg_v, [idx16])
                dp16 = jnp.where(cnt16 > sl, dr16 * d16, 0.0)
                plsc.store_scatter(dp_b, [flat16], dp16)
            return _

        lax.fori_loop(0, ROWS_PER_W // 16, dp_outer, None)

        pltpu.sync_copy(idx_b, idx_hbm.at[pl.ds(rlo * CAP, ROWS_PER_W * CAP)])
        pltpu.sync_copy(dp_b, dp_hbm.at[pl.ds(rlo * CAP, ROWS_PER_W * CAP)])

    return k(rows_pad, cols_pad, deg_flat)


def _gather_rows(nbr_idx_flat, h):
    OUTER = CH_PER_W // NBUF
    TAIL = CH_PER_W - OUTER * NBUF

    @functools.partial(
        pl.kernel,
        mesh=_mesh(),
        out_type=jax.ShapeDtypeStruct((FLAT_PAD, D), jnp.float32),
        scratch_types=[
            pltpu.VMEM((CH_PER_W * GCHUNK,), jnp.int32),
            [pltpu.VMEM((GCHUNK, D), jnp.float32) for _ in range(NBUF)],
            [pltpu.SemaphoreType.DMA for _ in range(NBUF)],
        ],
        compiler_params=_SC_PARAMS,
        name="sc_gather_rows",
    )
    def k(idx_hbm, h_hbm, out_hbm, idx_v, bufs, sems):
        wid = _wid()
        base = wid * CH_PER_W

        pltpu.sync_copy(idx_hbm.at[pl.ds(base * GCHUNK, CH_PER_W * GCHUNK)],
                        idx_v)

        def gather(i, b):
            pltpu.async_copy(
                h_hbm.at[idx_v.at[pl.ds(i * GCHUNK, GCHUNK)]], bufs[b],
                sems[b])

        def drain(i, b):
            pltpu.make_async_copy(
                h_hbm.at[idx_v.at[pl.ds(i * GCHUNK, GCHUNK)]], bufs[b],
                sems[b]).wait()
            pltpu.sync_copy(bufs[b],
                            out_hbm.at[pl.ds((base + i) * GCHUNK, GCHUNK)])

        for b in range(NBUF):
            gather(b, b)

        def outer(o, _):
            for b in range(NBUF):
                i = o * NBUF + b
                drain(i, b)
                gather(i + NBUF, b)
            return _

        lax.fori_loop(0, OUTER - 1, outer, None)
        for b in range(NBUF):
            i = (OUTER - 1) * NBUF + b
            drain(i, b)
            if b < TAIL:
                gather(i + NBUF, b)
        for b in range(TAIL):
            i = OUTER * NBUF + b
            drain(i, b)

    return k(nbr_idx_flat, h)


def _medoid_aggregate(hn, dp, b, eyeT, R, w2=None):
    RC = R * CAP

    def body(hn_ref, dp_ref, b_ref, eyeT_ref, *rest):
        f32 = jnp.float32

        def dot(a, bb, dims):
            return lax.dot_general(a, bb, (dims, ((), ())),
                                   preferred_element_type=f32)

        dpb = dp_ref[...]
        w = jnp.where(dpb > 0.0, lax.rsqrt(jnp.maximum(dpb, 1e-30)), 0.0)
        rs = jnp.sum(w, axis=1, keepdims=True)

        eyeT = eyeT_ref[...]

        gs = []
        for r in range(R):
            hr = hn_ref[r]
            gs.append(dot(hr, hr, ((1,), (1,))))
        g_all = jnp.concatenate(gs, axis=0)

        def bexp(x):
            return jnp.broadcast_to(x[:, None, :], (R, CAP, CAP)).reshape(RC, CAP)

        def bsum(x, n):
            return jnp.sum(x.reshape(R, CAP, n), axis=1)

        gd = g_all * eyeT
        sq_col = jnp.sum(gd, axis=1, keepdims=True)
        sq_rows = bsum(gd, CAP)
        d2 = jnp.maximum(sq_col + bexp(sq_rows) - 2.0 * g_all, 0.0)
        dist = jnp.sqrt(d2 + 1e-12)
        dist = jnp.where(eyeT > 0.5, 1e-6, dist)

        w_col = jnp.sum(bexp(w) * eyeT, axis=1, keepdims=True)
        dk = bsum(dist * w_col, CAP)

        valid = dpb > 0.0
        z = -dk / rs
        e = jnp.where(valid, jnp.exp(z), 0.0)
        uw = e * w
        s = jnp.sum(uw, axis=1, keepdims=True)
        wgt = uw / s

        w2_col = jnp.sum(bexp(wgt) * eyeT, axis=1, keepdims=True)
        hc = hn_ref[...].reshape(RC, D)
        out = bsum(hc * w2_col, D)
        ob = rs * out + b_ref[...]
        relu = jnp.maximum(ob, 0.0)
        if w2 is None:
            o_ref = rest[0]
            o_ref[...] = relu
        else:
            w2_ref, o_ref = rest
            o_ref[...] = dot(relu, w2_ref[...], ((1,), (0,)))

    return pl.pallas_call(
        body,
        grid=(N // R,),
        in_specs=[
            pl.BlockSpec((R, CAP, D), lambda i: (i, 0, 0)),
            pl.BlockSpec((R, CAP), lambda i: (i, 0)),
            pl.BlockSpec((1, D), lambda i: (0, 0)),
            pl.BlockSpec((R * CAP, CAP), lambda i: (0, 0)),
        ] + ([] if w2 is None else [pl.BlockSpec((D, D), lambda i: (0, 0))]),
        out_specs=pl.BlockSpec((R, D), lambda i: (i, 0)),
        out_shape=jax.ShapeDtypeStruct((N, D), jnp.float32),
        name="tc_medoid_aggregate",
    )(*((hn, dp, b, eyeT) if w2 is None else (hn, dp, b, eyeT, w2)))


def kernel(x, edge_index, W1, b1, W2, b2):
    pad = jnp.full((E_PAD - E,), SENT, jnp.int32)
    rows_pad = jnp.concatenate([edge_index[0].astype(jnp.int32), pad])
    cols_pad = jnp.concatenate([edge_index[1].astype(jnp.int32), pad])

    degp = _deg_partial(cols_pad)
    h1, deg = _matmul_deg(x, W1, degp.reshape(NW, N_PAD // 128, 128))
    deg_flat = deg.reshape(N_PAD)

    nbr_idx, nbr_dp = _build_lists(rows_pad, cols_pad, deg_flat)
    dp2d = nbr_dp.reshape(N_PAD, CAP)
    idx_gather = nbr_idx[:FLAT_PAD]

    b1r = b1.reshape(1, D)
    b2r = b2.reshape(1, D)

    R = 40
    eyeT = (jnp.arange(R * CAP, dtype=jnp.int32)[:, None] % CAP
            == jnp.arange(CAP, dtype=jnp.int32)[None, :]).astype(jnp.float32)

    hn1 = _gather_rows(idx_gather, h1).reshape(FLAT_PAD // CAP, CAP, D)
    h2 = _medoid_aggregate(hn1, dp2d, b1r, eyeT, R, w2=W2)

    hn2 = _gather_rows(idx_gather, h2).reshape(FLAT_PAD // CAP, CAP, D)
    o2 = _medoid_aggregate(hn2, dp2d, b2r, eyeT, R)
    return o2
